# baseline (speedup 1.0000x reference)
import functools

import jax
import jax.numpy as jnp
from jax import lax
from jax.experimental import pallas as pl
from jax.experimental.pallas import tpu as pltpu
from jax.experimental.pallas import tpu_sc as plsc

_LANES = 16
_N_CLASSES = 10
_MAX_DEPTH = 12
_UNROLL = 4


def _transpose_body(x_ref, o_ref):
    o_ref[...] = x_ref[...].T


def _transpose(x):
    b, f = x.shape
    bb = 512
    return pl.pallas_call(
        _transpose_body,
        grid=(f // bb, b // bb),
        in_specs=[pl.BlockSpec((bb, bb), lambda i, j: (j, i))],
        out_specs=pl.BlockSpec((bb, bb), lambda i, j: (i, j)),
        out_shape=jax.ShapeDtypeStruct((f, b), x.dtype),
    )(x)


def _forest_sc(vT, tf, nf, thr, nl, nr, leaf):
    f, b = vT.shape
    t, s = tf.shape
    n = nf.shape[1]
    info = plsc.get_sparse_core_info()
    nc, ns = info.num_cores, info.num_subcores
    nw = nc * ns
    tpw = t // nw
    cb = b * _N_CLASSES
    step = _LANES * _UNROLL
    mesh = plsc.VectorSubcoreMesh(core_axis_name="c", subcore_axis_name="s")

    @functools.partial(
        pl.kernel,
        out_type=jax.ShapeDtypeStruct((nw, cb), jnp.float32),
        mesh=mesh,
        compiler_params=pltpu.CompilerParams(needs_layout_passes=False),
        scratch_types=[
            pltpu.VMEM((s,), jnp.int32),
            pltpu.VMEM((s, b), jnp.float32),
            pltpu.VMEM((n,), jnp.int32),
            pltpu.VMEM((n,), jnp.float32),
            pltpu.VMEM((n,), jnp.int32),
            pltpu.VMEM((n,), jnp.int32),
            pltpu.VMEM((n,), jnp.int32),
            pltpu.VMEM((cb,), jnp.float32),
            pltpu.VMEM((_LANES,), jnp.int32),
            pltpu.SemaphoreType.DMA,
        ],
    )
    def k(vT_h, tf_h, nf_h, thr_h, nl_h, nr_h, leaf_h, out_h,
          idx_v, sub_v, nf_v, thr_v, nl_v, nr_v, leaf_v, cnt_v, zero_v, sem):
        wid = lax.axis_index("s") * nc + lax.axis_index("c")
        iota = lax.iota(jnp.int32, _LANES)
        zeros = jnp.zeros((_LANES,), jnp.float32)
        ones = jnp.ones((_LANES,), jnp.float32)

        def zero_body(i, carry):
            cnt_v[pl.ds(i * _LANES, _LANES)] = zeros
            return carry

        lax.fori_loop(0, cb // _LANES, zero_body, 0)
        zero_v[...] = jnp.zeros((_LANES,), jnp.int32)

        def tree_body(kk, carry):
            tree = wid * tpw + kk
            pltpu.sync_copy(tf_h.at[tree], idx_v)
            copies = [
                pltpu.async_copy(vT_h.at[idx_v], sub_v, sem),
                pltpu.async_copy(nf_h.at[tree], nf_v, sem),
                pltpu.async_copy(thr_h.at[tree], thr_v, sem),
                pltpu.async_copy(nl_h.at[tree], nl_v, sem),
                pltpu.async_copy(nr_h.at[tree], nr_v, sem),
                pltpu.async_copy(leaf_h.at[tree], leaf_v, sem),
            ]
            for c in copies:
                c.wait()

            def chunk_body(i, ccarry):
                base = i * step
                cols = [base + u * _LANES + iota for u in range(_UNROLL)]
                nodes = [zero_v[...] for _ in range(_UNROLL)]
                for _ in range(_MAX_DEPTH):
                    feats = [plsc.load_gather(nf_v, [nd]) for nd in nodes]
                    ths = [plsc.load_gather(thr_v, [nd]) for nd in nodes]
                    lts = [plsc.load_gather(nl_v, [nd]) for nd in nodes]
                    rts = [plsc.load_gather(nr_v, [nd]) for nd in nodes]
                    vals = [plsc.load_gather(sub_v, [fe, co])
                            for fe, co in zip(feats, cols)]
                    nodes = [jnp.where(v < th, lt, rt)
                             for v, th, lt, rt in zip(vals, ths, lts, rts)]
                for u in range(_UNROLL):
                    pred = plsc.load_gather(leaf_v, [nodes[u]])
                    plsc.addupdate_scatter(cnt_v, [pred * b + cols[u]], ones)
                return ccarry

            lax.fori_loop(0, b // step, chunk_body, 0)
            return carry

        lax.fori_loop(0, tpw, tree_body, 0)
        pltpu.sync_copy(cnt_v, out_h.at[wid])

    return k(vT, tf, nf, thr, nl, nr, leaf)


def _reduce(parts, n_trees):
    nw, ncls, b = parts.shape
    scale = 1.0 / n_trees

    def body(c_ref, probs_ref, cls_ref):
        c = c_ref[...]
        tot = jnp.sum(c, axis=0)
        probs_ref[...] = tot * scale
        idx0 = lax.broadcasted_iota(jnp.int32, tot.shape, 0)
        mx = jnp.max(tot, axis=0, keepdims=True)
        cand = jnp.where(tot == mx, idx0, ncls)
        cls_ref[...] = jnp.min(cand, axis=0, keepdims=True)

    return pl.pallas_call(
        body,
        out_shape=(
            jax.ShapeDtypeStruct((ncls, b), jnp.float32),
            jax.ShapeDtypeStruct((1, b), jnp.int32),
        ),
    )(parts)


def kernel(vector, node_threshold, trees_features, node_feature,
           node_left, node_right, leaf_label):
    b, f = vector.shape
    t, n = node_feature.shape
    vT = _transpose(vector)
    parts = _forest_sc(vT, trees_features, node_feature, node_threshold,
                       node_left, node_right, leaf_label)
    probs_t, cls = _reduce(parts.reshape(-1, _N_CLASSES, b), t)
    return cls.reshape(b), probs_t.T


# speedup vs baseline: 1.1305x; 1.1305x over previous
import functools

import jax
import jax.numpy as jnp
from jax import lax
from jax.experimental import pallas as pl
from jax.experimental.pallas import tpu as pltpu
from jax.experimental.pallas import tpu_sc as plsc

_LANES = 16
_N_CLASSES = 10
_MAX_DEPTH = 12
_UNROLL = 4


def _transpose_body(x_ref, o_ref):
    o_ref[0, :, :] = x_ref[...].T


def _transpose_halves(x):
    b, f = x.shape
    bb = 512
    nh = b // bb
    return pl.pallas_call(
        _transpose_body,
        grid=(f // bb, nh),
        in_specs=[pl.BlockSpec((bb, bb), lambda i, j: (j, i))],
        out_specs=pl.BlockSpec((1, bb, bb), lambda i, j: (j, i, 0)),
        out_shape=jax.ShapeDtypeStruct((nh, f, bb), x.dtype),
    )(x)


def _forest_sc(vT2, tf, nf, thr, nl, nr, leaf):
    nh, f, bh = vT2.shape
    b = nh * bh
    t, s = tf.shape
    n = nf.shape[1]
    info = plsc.get_sparse_core_info()
    nc, ns = info.num_cores, info.num_subcores
    nw = nc * ns
    tpw = t // nw
    step = _LANES * _UNROLL
    mesh = plsc.VectorSubcoreMesh(core_axis_name="c", subcore_axis_name="s")

    @functools.partial(
        pl.kernel,
        out_type=jax.ShapeDtypeStruct((nw, _N_CLASSES, b), jnp.float32),
        mesh=mesh,
        compiler_params=pltpu.CompilerParams(needs_layout_passes=False),
        scratch_types=[
            pltpu.VMEM((s,), jnp.int32),
            pltpu.VMEM((s,), jnp.int32),
            pltpu.VMEM((s, bh), jnp.float32),
            pltpu.VMEM((s, bh), jnp.float32),
            pltpu.VMEM((n,), jnp.int32),
            pltpu.VMEM((n,), jnp.float32),
            pltpu.VMEM((n,), jnp.int32),
            pltpu.VMEM((n,), jnp.int32),
            pltpu.VMEM((n,), jnp.int32),
            pltpu.VMEM((_N_CLASSES, b), jnp.float32),
            pltpu.VMEM((_LANES,), jnp.int32),
            pltpu.SemaphoreType.DMA,
            pltpu.SemaphoreType.DMA,
            pltpu.SemaphoreType.DMA,
        ],
    )
    def k(vT2_h, tf_h, nf_h, thr_h, nl_h, nr_h, leaf_h, out_h,
          idx0_v, idx1_v, sub0_v, sub1_v,
          nf_v, thr_v, nl_v, nr_v, leaf_v, cnt_v, zero_v,
          sem_t, sem_s0, sem_s1):
        wid = lax.axis_index("s") * nc + lax.axis_index("c")
        iota = lax.iota(jnp.int32, _LANES)
        zeros = jnp.zeros((_LANES,), jnp.float32)
        ones = jnp.ones((_LANES,), jnp.float32)
        idx_bufs = (idx0_v, idx1_v)
        sub_bufs = (sub0_v, sub1_v)
        sub_sems = (sem_s0, sem_s1)

        def tab_copies(tree):
            return [
                pltpu.make_async_copy(nf_h.at[tree], nf_v, sem_t),
                pltpu.make_async_copy(thr_h.at[tree], thr_v, sem_t),
                pltpu.make_async_copy(nl_h.at[tree], nl_v, sem_t),
                pltpu.make_async_copy(nr_h.at[tree], nr_v, sem_t),
                pltpu.make_async_copy(leaf_h.at[tree], leaf_v, sem_t),
            ]

        def sub_copy(h, idx_v):
            return pltpu.make_async_copy(
                vT2_h.at[h].at[idx_v], sub_bufs[h], sub_sems[h])

        for r in range(_N_CLASSES):
            def zero_body(i, carry, r=r):
                cnt_v[r, pl.ds(i * _LANES, _LANES)] = zeros
                return carry
            lax.fori_loop(0, b // _LANES, zero_body, 0)
        zero_v[...] = jnp.zeros((_LANES,), jnp.int32)

        def run_half(h, sub_v):
            def chunk_body(i, ccarry):
                base = i * step
                lcols = [base + u * _LANES + iota for u in range(_UNROLL)]
                nodes = [zero_v[...] for _ in range(_UNROLL)]
                for _ in range(_MAX_DEPTH):
                    feats = [plsc.load_gather(nf_v, [nd]) for nd in nodes]
                    ths = [plsc.load_gather(thr_v, [nd]) for nd in nodes]
                    lts = [plsc.load_gather(nl_v, [nd]) for nd in nodes]
                    rts = [plsc.load_gather(nr_v, [nd]) for nd in nodes]
                    vals = [plsc.load_gather(sub_v, [fe, co])
                            for fe, co in zip(feats, lcols)]
                    nodes = [jnp.where(v < th, lt, rt)
                             for v, th, lt, rt in zip(vals, ths, lts, rts)]
                for u in range(_UNROLL):
                    pred = plsc.load_gather(leaf_v, [nodes[u]])
                    plsc.addupdate_scatter(
                        cnt_v, [pred, h * bh + lcols[u]], ones)
                return ccarry

            lax.fori_loop(0, bh // step, chunk_body, 0)

        tree0 = wid * tpw
        pltpu.sync_copy(tf_h.at[tree0], idx0_v)
        for c in tab_copies(tree0):
            c.start()
        sub_copy(0, idx0_v).start()

        def pair_body(kk, carry):
            for par in range(2):
                tree = wid * tpw + 2 * kk + par
                nxt = jnp.minimum(tree + 1, t - 1)
                idx_cur = idx_bufs[par]
                idx_nxt = idx_bufs[1 - par]
                for c in tab_copies(tree):
                    c.wait()
                sub_copy(0, idx_cur).wait()
                sub_copy(1, idx_cur).start()
                run_half(0, sub0_v)
                sub_copy(1, idx_cur).wait()
                pltpu.sync_copy(tf_h.at[nxt], idx_nxt)
                sub_copy(0, idx_nxt).start()
                run_half(1, sub1_v)
                for c in tab_copies(nxt):
                    c.start()
            return carry

        lax.fori_loop(0, tpw // 2, pair_body, 0)

        last = jnp.minimum(wid * tpw + tpw, t) - 1
        for c in tab_copies(last):
            c.wait()
        sub_copy(0, idx0_v).wait()

        pltpu.sync_copy(cnt_v, out_h.at[wid])

    return k(vT2, tf, nf, thr, nl, nr, leaf)


def _reduce(parts, n_trees):
    nw, ncls, b = parts.shape
    scale = 1.0 / n_trees

    def body(c_ref, probs_ref, cls_ref):
        c = c_ref[...]
        tot = jnp.sum(c, axis=0)
        probs_ref[...] = tot * scale
        idx0 = lax.broadcasted_iota(jnp.int32, tot.shape, 0)
        mx = jnp.max(tot, axis=0, keepdims=True)
        cand = jnp.where(tot == mx, idx0, ncls)
        cls_ref[...] = jnp.min(cand, axis=0, keepdims=True)

    return pl.pallas_call(
        body,
        out_shape=(
            jax.ShapeDtypeStruct((ncls, b), jnp.float32),
            jax.ShapeDtypeStruct((1, b), jnp.int32),
        ),
    )(parts)


def kernel(vector, node_threshold, trees_features, node_feature,
           node_left, node_right, leaf_label):
    b, f = vector.shape
    t, n = node_feature.shape
    vT2 = _transpose_halves(vector)
    parts = _forest_sc(vT2, trees_features, node_feature, node_threshold,
                       node_left, node_right, leaf_label)
    probs_t, cls = _reduce(parts, t)
    return cls.reshape(b), probs_t.T
